# Initial kernel scaffold; baseline (speedup 1.0000x reference)
import jax
import jax.numpy as jnp
from jax.experimental import pallas as pl


def kernel(predictions, relevance_scores):
    raise NotImplementedError("write your pallas kernel here")


# speedup vs baseline: 2.8862x; 2.8862x over previous
import jax
import jax.numpy as jnp
from jax import lax
from jax.experimental import pallas as pl
from jax.experimental.pallas import tpu as pltpu
from jax.experimental.pallas import tpu_sc as plsc

_K = 10
_TEMPERATURE = 1.0
_ROWS = 16384
_N = 1000
_LANES = 16
_NCHUNK = 63
_NPAD = _NCHUNK * _LANES
_NWORKERS = 32
_RPW = _ROWS // _NWORKERS


def _allreduce(v, op, lanes):
  for sh in (8, 4, 2, 1):
    v = op(v, v[jnp.bitwise_xor(lanes, sh)])
  return v


def _body(p_hbm, r_hbm, w_hbm, out_hbm, pbuf, rbuf, wbuf, obuf):
  wid = lax.axis_index("s") * 2 + lax.axis_index("c")
  base = wid * _RPW

  lanes = lax.iota(jnp.int32, _LANES)
  flanes = lanes.astype(jnp.float32)

  pltpu.sync_copy(w_hbm, wbuf)
  wvec = wbuf[...]

  pbuf[pl.ds(_N - 8, _LANES)] = jnp.where(flanes < 8.0, 0.0, -jnp.inf)
  rbuf[pl.ds(_N - 8, _LANES)] = jnp.where(flanes < 8.0, 0.0, -1.0)

  def row_step(r, acc):
    row = base + r
    pltpu.sync_copy(p_hbm.at[pl.ds(row * _N, _N)], pbuf.at[pl.ds(0, _N)])
    pltpu.sync_copy(r_hbm.at[pl.ds(row * _N, _N)], rbuf.at[pl.ds(0, _N)])

    def mx_step(c, m):
      return jnp.maximum(m, pbuf[pl.ds(c * _LANES, _LANES)])

    mvec = lax.fori_loop(0, _NCHUNK, mx_step,
                         jnp.full((_LANES,), -jnp.inf, jnp.float32))
    pmax = _allreduce(mvec, jnp.maximum, lanes)

    def se_step(c, s):
      return s + jnp.exp(pbuf[pl.ds(c * _LANES, _LANES)] - pmax)

    svec = lax.fori_loop(0, _NCHUNK, se_step, jnp.zeros((_LANES,), jnp.float32))
    sumexp = _allreduce(svec, jnp.add, lanes)

    def tk_step(c, carry):
      run_v, run_i = carry
      cv = rbuf[pl.ds(c * _LANES, _LANES)]
      ci = c * _LANES + lanes
      cvs, cis = plsc.sort_key_val(cv, ci, descending=False)
      take = cvs > run_v
      hv = jnp.where(take, cvs, run_v)
      hi = jnp.where(take, cis, run_i)
      nv, ni = plsc.sort_key_val(hv, hi, descending=True)
      return (nv, ni)

    run_v, run_i = lax.fori_loop(
        0, _NCHUNK, tk_step,
        (jnp.full((_LANES,), -2.0, jnp.float32),
         jnp.zeros((_LANES,), jnp.int32)))

    rank = jnp.zeros((_LANES,), jnp.int32)
    for j in range(_LANES):
      jv = jnp.full((_LANES,), j, jnp.int32)
      bv = run_v[jv]
      bi = run_i[jv]
      beats = (bv > run_v) | ((bv == run_v) & (bi < run_i))
      rank = rank + beats.astype(jnp.int32)

    wr = wvec[rank]
    pv = plsc.load_gather(pbuf, [run_i])
    soft = jnp.exp(pv - pmax) / sumexp
    dcg = _allreduce(run_v * soft * wr, jnp.add, lanes)
    idcg = _allreduce(run_v * wr, jnp.add, lanes)
    return acc + dcg / (idcg + 1e-8)

  acc = lax.fori_loop(0, _RPW, row_step, jnp.zeros((_LANES,), jnp.float32))
  obuf[...] = acc
  pltpu.sync_copy(obuf, out_hbm.at[wid])


@jax.jit
def kernel(predictions, relevance_scores):
  positions = jnp.arange(_LANES, dtype=jnp.float32)
  wtable = jnp.where(positions < _K,
                     1.0 / jnp.log2(positions + 2.0), 0.0).astype(jnp.float32)

  mesh = plsc.VectorSubcoreMesh(core_axis_name="c", subcore_axis_name="s")
  partials = pl.kernel(
      _body,
      out_type=jax.ShapeDtypeStruct((_NWORKERS, _LANES), jnp.float32),
      mesh=mesh,
      compiler_params=pltpu.CompilerParams(needs_layout_passes=False),
      scratch_types=[
          pltpu.VMEM((_NPAD,), jnp.float32),
          pltpu.VMEM((_NPAD,), jnp.float32),
          pltpu.VMEM((_LANES,), jnp.float32),
          pltpu.VMEM((_LANES,), jnp.float32),
      ],
  )(
      (predictions / _TEMPERATURE).reshape(-1),
      relevance_scores.reshape(-1),
      wtable,
  )
  return -jnp.sum(partials[:, 0]) / _ROWS


# speedup vs baseline: 6.1151x; 2.1187x over previous
import jax
import jax.numpy as jnp
from jax import lax
from jax.experimental import pallas as pl
from jax.experimental.pallas import tpu as pltpu
from jax.experimental.pallas import tpu_sc as plsc

_K = 10
_TEMPERATURE = 1.0
_ROWS = 16384
_N = 1000
_LANES = 16
_NCHUNK = 63
_NWORKERS = 32
_RPW = _ROWS // _NWORKERS
_BATCH = 16
_NBATCH = _RPW // _BATCH
_BUFLEN = _BATCH * _N + 8
_NSTREAM = 4


def _allreduce(v, op, lanes):
  for sh in (8, 4, 2, 1):
    v = op(v, v[jnp.bitwise_xor(lanes, sh)])
  return v


def _merge_desc(av, ai, bv, bi, rev):
  bvr = bv[rev]
  bir = bi[rev]
  take = bvr > av
  hv = jnp.where(take, bvr, av)
  hi = jnp.where(take, bir, ai)
  nv, ni = plsc.sort_key_val(hv, hi, descending=True)
  return nv, ni


def _body(p_hbm, r_hbm, w_hbm, out_hbm,
          pbuf0, pbuf1, rbuf0, rbuf1, wbuf, obuf, sems):
  pbufs = (pbuf0, pbuf1)
  rbufs = (rbuf0, rbuf1)
  wid = lax.axis_index("s") * 2 + lax.axis_index("c")
  base = wid * _RPW

  lanes = lax.iota(jnp.int32, _LANES)
  rev = 15 - lanes
  tail_mask = lanes < 8
  minf = jnp.full((_LANES,), -jnp.inf, jnp.float32)
  neg1 = jnp.full((_LANES,), -1.0, jnp.float32)

  pltpu.sync_copy(w_hbm, wbuf)
  wvec = wbuf[...]

  def copy_batch(j, par):
    off = (base + j * _BATCH) * _N
    n = _BATCH * _N
    pltpu.async_copy(p_hbm.at[pl.ds(off, n)], pbufs[par].at[pl.ds(0, n)],
                     sems.at[2 * par])
    pltpu.async_copy(r_hbm.at[pl.ds(off, n)], rbufs[par].at[pl.ds(0, n)],
                     sems.at[2 * par + 1])

  def wait_batch(j, par):
    off = (base + j * _BATCH) * _N
    n = _BATCH * _N
    pltpu.make_async_copy(p_hbm.at[pl.ds(off, n)],
                          pbufs[par].at[pl.ds(0, n)],
                          sems.at[2 * par]).wait()
    pltpu.make_async_copy(r_hbm.at[pl.ds(off, n)],
                          rbufs[par].at[pl.ds(0, n)],
                          sems.at[2 * par + 1]).wait()

  def process_batch(par, acc0):
    pb = pbufs[par]
    rb = rbufs[par]

    def row_step(r, acc):
      o = r * _N

      m = minf
      for c in range(_NCHUNK):
        v = pb[pl.ds(o + c * _LANES, _LANES)]
        if c == _NCHUNK - 1:
          v = jnp.where(tail_mask, v, minf)
        m = jnp.maximum(m, v)
      pmax = _allreduce(m, jnp.maximum, lanes)

      s = jnp.zeros((_LANES,), jnp.float32)
      for c in range(_NCHUNK):
        e = jnp.exp(pb[pl.ds(o + c * _LANES, _LANES)] - pmax)
        if c == _NCHUNK - 1:
          e = jnp.where(tail_mask, e, 0.0)
        s = s + e
      sumexp = _allreduce(s, jnp.add, lanes)

      run_v = [jnp.full((_LANES,), -2.0, jnp.float32)] * _NSTREAM
      run_i = [jnp.zeros((_LANES,), jnp.int32)] * _NSTREAM
      for c in range(_NCHUNK):
        st = c % _NSTREAM
        cv = rb[pl.ds(o + c * _LANES, _LANES)]
        if c == _NCHUNK - 1:
          cv = jnp.where(tail_mask, cv, neg1)
        ci = c * _LANES + lanes
        cvs, cis = plsc.sort_key_val(cv, ci, descending=False)
        take = cvs > run_v[st]
        hv = jnp.where(take, cvs, run_v[st])
        hi = jnp.where(take, cis, run_i[st])
        nv, ni = plsc.sort_key_val(hv, hi, descending=True)
        run_v[st], run_i[st] = nv, ni

      v01, i01 = _merge_desc(run_v[0], run_i[0], run_v[1], run_i[1], rev)
      v23, i23 = _merge_desc(run_v[2], run_i[2], run_v[3], run_i[3], rev)
      top_v, top_i = _merge_desc(v01, i01, v23, i23, rev)

      rank = jnp.zeros((_LANES,), jnp.int32)
      for j in range(_LANES):
        jv = jnp.full((_LANES,), j, jnp.int32)
        bv = top_v[jv]
        bi = top_i[jv]
        beats = (bv > top_v) | ((bv == top_v) & (bi < top_i))
        rank = rank + beats.astype(jnp.int32)

      wr = wvec[rank]
      pv = plsc.load_gather(pb, [top_i + o])
      soft = jnp.exp(pv - pmax) / sumexp
      dcg = _allreduce(top_v * soft * wr, jnp.add, lanes)
      idcg = _allreduce(top_v * wr, jnp.add, lanes)
      return acc + dcg / (idcg + 1e-8)

    return lax.fori_loop(0, _BATCH, row_step, acc0)

  copy_batch(0, 0)

  def pair_step(i, acc):
    wait_batch(2 * i, 0)
    copy_batch(2 * i + 1, 1)
    acc = process_batch(0, acc)
    wait_batch(2 * i + 1, 1)

    @pl.when(i + 1 < _NBATCH // 2)
    def _():
      copy_batch(2 * i + 2, 0)

    return process_batch(1, acc)

  acc = lax.fori_loop(0, _NBATCH // 2, pair_step,
                      jnp.zeros((_LANES,), jnp.float32))
  obuf[...] = acc
  pltpu.sync_copy(obuf, out_hbm.at[wid])


@jax.jit
def kernel(predictions, relevance_scores):
  positions = jnp.arange(_LANES, dtype=jnp.float32)
  wtable = jnp.where(positions < _K,
                     1.0 / jnp.log2(positions + 2.0), 0.0).astype(jnp.float32)

  mesh = plsc.VectorSubcoreMesh(core_axis_name="c", subcore_axis_name="s")
  partials = pl.kernel(
      _body,
      out_type=jax.ShapeDtypeStruct((_NWORKERS, _LANES), jnp.float32),
      mesh=mesh,
      compiler_params=pltpu.CompilerParams(needs_layout_passes=False),
      scratch_types=[
          pltpu.VMEM((_BUFLEN,), jnp.float32),
          pltpu.VMEM((_BUFLEN,), jnp.float32),
          pltpu.VMEM((_BUFLEN,), jnp.float32),
          pltpu.VMEM((_BUFLEN,), jnp.float32),
          pltpu.VMEM((_LANES,), jnp.float32),
          pltpu.VMEM((_LANES,), jnp.float32),
          pltpu.SemaphoreType.DMA((4,)),
      ],
  )(
      (predictions / _TEMPERATURE).reshape(-1),
      relevance_scores.reshape(-1),
      wtable,
  )
  return -jnp.sum(partials[:, 0]) / _ROWS


# speedup vs baseline: 6.9041x; 1.1290x over previous
import jax
import jax.numpy as jnp
from jax import lax
from jax.experimental import pallas as pl
from jax.experimental.pallas import tpu as pltpu
from jax.experimental.pallas import tpu_sc as plsc

_K = 10
_TEMPERATURE = 1.0
_ROWS = 16384
_N = 1000
_LANES = 16
_NCHUNK = 63
_NWORKERS = 32
_RPW = _ROWS // _NWORKERS
_BATCH = 16
_NBATCH = _RPW // _BATCH
_NSTREAM = 8


def _allreduce(v, op, lanes):
  for sh in (8, 4, 2, 1):
    v = op(v, v[jnp.bitwise_xor(lanes, sh)])
  return v


def _merge_desc(av, ai, bv, bi, rev):
  bvr = bv[rev]
  bir = bi[rev]
  take = bvr > av
  hv = jnp.where(take, bvr, av)
  hi = jnp.where(take, bir, ai)
  nv, ni = plsc.sort_key_val(hv, hi, descending=True)
  return nv, ni


def _body(p_hbm, r_hbm, w_hbm, out_hbm,
          pbuf0, pbuf1, rbuf0, rbuf1, wbuf, obuf, sems):
  pbufs = (pbuf0, pbuf1)
  rbufs = (rbuf0, rbuf1)
  wid = lax.axis_index("s") * 2 + lax.axis_index("c")
  base = wid * _RPW

  lanes = lax.iota(jnp.int32, _LANES)
  rev = 15 - lanes
  tail_mask = lanes < 8
  minf = jnp.full((_LANES,), -jnp.inf, jnp.float32)
  neg1 = jnp.full((_LANES,), -1.0, jnp.float32)

  cols = [c * _LANES + lanes for c in range(_NCHUNK - 1)]
  cols.append(jnp.minimum((_NCHUNK - 1) * _LANES + lanes, _N - 1))

  pltpu.sync_copy(w_hbm, wbuf)
  wvec = wbuf[...]

  def copy_batch(j, par):
    r0 = base + j * _BATCH
    pltpu.async_copy(p_hbm.at[pl.ds(r0, _BATCH), :], pbufs[par],
                     sems.at[2 * par])
    pltpu.async_copy(r_hbm.at[pl.ds(r0, _BATCH), :], rbufs[par],
                     sems.at[2 * par + 1])

  def wait_batch(j, par):
    r0 = base + j * _BATCH
    pltpu.make_async_copy(p_hbm.at[pl.ds(r0, _BATCH), :], pbufs[par],
                          sems.at[2 * par]).wait()
    pltpu.make_async_copy(r_hbm.at[pl.ds(r0, _BATCH), :], rbufs[par],
                          sems.at[2 * par + 1]).wait()

  def process_batch(par, acc0):
    pb = pbufs[par]
    rb = rbufs[par]

    def row_step(r, acc):
      rsplat = jnp.full((_LANES,), 0, jnp.int32) + r

      m = minf
      for c in range(_NCHUNK):
        v = plsc.load_gather(pb, [rsplat, cols[c]])
        if c == _NCHUNK - 1:
          v = jnp.where(tail_mask, v, minf)
        m = jnp.maximum(m, v)
      pmax = _allreduce(m, jnp.maximum, lanes)

      s = jnp.zeros((_LANES,), jnp.float32)
      for c in range(_NCHUNK):
        e = jnp.exp(plsc.load_gather(pb, [rsplat, cols[c]]) - pmax)
        if c == _NCHUNK - 1:
          e = jnp.where(tail_mask, e, 0.0)
        s = s + e
      sumexp = _allreduce(s, jnp.add, lanes)

      run_v = [jnp.full((_LANES,), -2.0, jnp.float32)] * _NSTREAM
      run_i = [jnp.zeros((_LANES,), jnp.int32)] * _NSTREAM
      for c in range(_NCHUNK):
        st = c % _NSTREAM
        cv = plsc.load_gather(rb, [rsplat, cols[c]])
        if c == _NCHUNK - 1:
          cv = jnp.where(tail_mask, cv, neg1)
        ci = c * _LANES + lanes
        cvs, cis = plsc.sort_key_val(cv, ci, descending=False)
        take = cvs > run_v[st]
        hv = jnp.where(take, cvs, run_v[st])
        hi = jnp.where(take, cis, run_i[st])
        nv, ni = plsc.sort_key_val(hv, hi, descending=True)
        run_v[st], run_i[st] = nv, ni

      while len(run_v) > 1:
        nxt_v, nxt_i = [], []
        for a in range(0, len(run_v), 2):
          mv, mi = _merge_desc(run_v[a], run_i[a], run_v[a + 1], run_i[a + 1],
                               rev)
          nxt_v.append(mv)
          nxt_i.append(mi)
        run_v, run_i = nxt_v, nxt_i
      top_v, top_i = run_v[0], run_i[0]

      rank = jnp.zeros((_LANES,), jnp.int32)
      for j in range(_LANES):
        jv = jnp.full((_LANES,), j, jnp.int32)
        bv = top_v[jv]
        bi = top_i[jv]
        beats = (bv > top_v) | ((bv == top_v) & (bi < top_i))
        rank = rank + beats.astype(jnp.int32)

      wr = wvec[rank]
      pv = plsc.load_gather(pb, [rsplat, top_i])
      soft = jnp.exp(pv - pmax) / sumexp
      dcg = _allreduce(top_v * soft * wr, jnp.add, lanes)
      idcg = _allreduce(top_v * wr, jnp.add, lanes)
      return acc + dcg / (idcg + 1e-8)

    return lax.fori_loop(0, _BATCH, row_step, acc0)

  copy_batch(0, 0)

  def pair_step(i, acc):
    wait_batch(2 * i, 0)
    copy_batch(2 * i + 1, 1)
    acc = process_batch(0, acc)
    wait_batch(2 * i + 1, 1)

    @pl.when(i + 1 < _NBATCH // 2)
    def _():
      copy_batch(2 * i + 2, 0)

    return process_batch(1, acc)

  acc = lax.fori_loop(0, _NBATCH // 2, pair_step,
                      jnp.zeros((_LANES,), jnp.float32))
  obuf[...] = acc
  pltpu.sync_copy(obuf, out_hbm.at[wid])


@jax.jit
def kernel(predictions, relevance_scores):
  positions = jnp.arange(_LANES, dtype=jnp.float32)
  wtable = jnp.where(positions < _K,
                     1.0 / jnp.log2(positions + 2.0), 0.0).astype(jnp.float32)

  mesh = plsc.VectorSubcoreMesh(core_axis_name="c", subcore_axis_name="s")
  partials = pl.kernel(
      _body,
      out_type=jax.ShapeDtypeStruct((_NWORKERS, _LANES), jnp.float32),
      mesh=mesh,
      compiler_params=pltpu.CompilerParams(needs_layout_passes=False),
      scratch_types=[
          pltpu.VMEM((_BATCH, _N), jnp.float32),
          pltpu.VMEM((_BATCH, _N), jnp.float32),
          pltpu.VMEM((_BATCH, _N), jnp.float32),
          pltpu.VMEM((_BATCH, _N), jnp.float32),
          pltpu.VMEM((_LANES,), jnp.float32),
          pltpu.VMEM((_LANES,), jnp.float32),
          pltpu.SemaphoreType.DMA((4,)),
      ],
  )(
      predictions / _TEMPERATURE,
      relevance_scores,
      wtable,
  )
  return -jnp.sum(partials[:, 0]) / _ROWS
